# baseline (speedup 1.0000x reference)
import jax
import jax.numpy as jnp
from jax import lax
from jax.experimental import pallas as pl
from jax.experimental.pallas import tpu as pltpu

E = 8
TOPK = 2
D = 1024
FF = 1024
T = 256
DH = D // 2


def _combine_from_logits(g):
    iota = lax.broadcasted_iota(jnp.int32, g.shape, 1)
    m1 = jnp.max(g, axis=1, keepdims=True)
    i1 = jnp.min(jnp.where(g == m1, iota, E), axis=1, keepdims=True)
    mask1 = iota == i1
    g_rest = jnp.where(mask1, -jnp.inf, g)
    m2 = jnp.max(g_rest, axis=1, keepdims=True)
    i2 = jnp.min(jnp.where(g_rest == m2, iota, E), axis=1, keepdims=True)
    mask2 = iota == i2
    w1 = jax.nn.sigmoid(m1 - m2)
    w2 = 1.0 - w1
    return jnp.where(mask1, w1, 0.0) + jnp.where(mask2, w2, 0.0)


def _moe_body(x_ref, gating_ref, gate_ref, up_ref, d1_ref, d2_ref, out_ref,
              combine_ref):
    e = pl.program_id(0)
    nt = (((1,), (1,)), ((), ()))

    @pl.when(e == 0)
    def _():
        combine_ref[...] = _combine_from_logits(gating_ref[...])

    xb = x_ref[...].astype(jnp.bfloat16)
    gate_w = gate_ref[0].astype(jnp.bfloat16)
    up_w = up_ref[0].astype(jnp.bfloat16)
    gg = lax.dot_general(xb, gate_w, nt, preferred_element_type=jnp.float32)
    uu = lax.dot_general(xb, up_w, nt, preferred_element_type=jnp.float32)
    h = (gg * jax.nn.sigmoid(gg) * uu).astype(jnp.bfloat16)

    cm = combine_ref[...]
    sel = lax.broadcasted_iota(jnp.int32, cm.shape, 1) == e
    col = jnp.sum(jnp.where(sel, cm, 0.0), axis=1, keepdims=True)

    y1 = lax.dot_general(h, d1_ref[0].astype(jnp.bfloat16), nt,
                         preferred_element_type=jnp.float32)
    y2 = lax.dot_general(h, d2_ref[0].astype(jnp.bfloat16), nt,
                         preferred_element_type=jnp.float32)

    @pl.when(e == 0)
    def _():
        out_ref[:, :DH] = y1 * col
        out_ref[:, DH:] = y2 * col

    @pl.when(e != 0)
    def _():
        out_ref[:, :DH] += y1 * col
        out_ref[:, DH:] += y2 * col


@jax.jit
def kernel(x, gating_output, gate_up_proj, down_proj):
    out = pl.pallas_call(
        _moe_body,
        grid=(E,),
        in_specs=[
            pl.BlockSpec((T, D), lambda e: (0, 0)),
            pl.BlockSpec((T, E), lambda e: (0, 0)),
            pl.BlockSpec((1, FF, D), lambda e: (e, 0, 0)),
            pl.BlockSpec((1, FF, D), lambda e: (e, 1, 0)),
            pl.BlockSpec((1, DH, FF), lambda e: (e, 0, 0)),
            pl.BlockSpec((1, DH, FF), lambda e: (e, 1, 0)),
        ],
        out_specs=pl.BlockSpec((T, D), lambda e: (0, 0)),
        out_shape=jax.ShapeDtypeStruct((T, D), jnp.float32),
        scratch_shapes=[
            pltpu.VMEM((T, E), jnp.float32),
        ],
    )(x, gating_output, gate_up_proj, gate_up_proj, down_proj, down_proj)
    return out
